# baseline (speedup 1.0000x reference)
import functools

import jax
import jax.numpy as jnp
from jax import lax
from jax.experimental import pallas as pl
from jax.experimental.pallas import tpu as pltpu
from jax.experimental.pallas import tpu_sc as plsc

B = 16384
D = 16
H = 128
O = 32

_NC = 2
_NS = 16
_NW = _NC * _NS
B_PER_W = B // _NW
HALF = B_PER_W // 2
CHUNK = 128
L = 16
K = 64


@functools.cache
def _build_gather_sc():
    mesh = plsc.VectorSubcoreMesh(core_axis_name="c", subcore_axis_name="s")

    @functools.partial(
        pl.kernel,
        mesh=mesh,
        out_type=[
            jax.ShapeDtypeStruct((B, D), jnp.float32),
            jax.ShapeDtypeStruct((B, D), jnp.float32),
        ],
        scratch_types=[
            pltpu.VMEM((B_PER_W,), jnp.int32),
            pltpu.VMEM((B_PER_W,), jnp.int32),
            pltpu.VMEM((HALF, D), jnp.float32),
            pltpu.VMEM((HALF, D), jnp.float32),
            pltpu.SemaphoreType.DMA,
            pltpu.SemaphoreType.DMA,
        ],
        compiler_params=pltpu.CompilerParams(needs_layout_passes=False),
    )
    def gather_sc(sids, bids, bat, bowl, out_bat, out_bowl,
                  sid_v, bid_v, rows_a, rows_b, sem_a, sem_b):
        wid = lax.axis_index("s") * _NC + lax.axis_index("c")
        base = wid * B_PER_W
        pltpu.sync_copy(sids.at[pl.ds(base, B_PER_W)], sid_v)
        pltpu.sync_copy(bids.at[pl.ds(base, B_PER_W)], bid_v)

        for h in range(2):
            @pl.loop(0, HALF, step=K)
            def _(p0):
                copies = []
                for g in range(K // L):
                    qa = sid_v[pl.ds(h * HALF + p0 + g * L, L)]
                    qb = bid_v[pl.ds(h * HALF + p0 + g * L, L)]
                    for i in range(L):
                        p = p0 + g * L + i
                        copies.append(pltpu.async_copy(
                            bat.at[qa[i]], rows_a.at[p], sem_a))
                        copies.append(pltpu.async_copy(
                            bowl.at[qb[i]], rows_b.at[p], sem_b))
                for c in copies:
                    c.wait()

            for j in range(HALF // CHUNK):
                pltpu.sync_copy(
                    rows_a.at[pl.ds(j * CHUNK, CHUNK)],
                    out_bat.at[pl.ds(base + h * HALF + j * CHUNK, CHUNK)])
                pltpu.sync_copy(
                    rows_b.at[pl.ds(j * CHUNK, CHUNK)],
                    out_bowl.at[pl.ds(base + h * HALF + j * CHUNK, CHUNK)])

    return gather_sc


NCH = B_PER_W // CHUNK


@functools.cache
def _build_gather_sc16():
    mesh = plsc.VectorSubcoreMesh(core_axis_name="c", subcore_axis_name="s")

    @functools.partial(
        pl.kernel,
        mesh=mesh,
        out_type=[
            jax.ShapeDtypeStruct((B, D), jnp.bfloat16),
            jax.ShapeDtypeStruct((B, D), jnp.bfloat16),
        ],
        scratch_types=[
            pltpu.VMEM((NCH, CHUNK), jnp.int32),
            pltpu.VMEM((NCH, CHUNK), jnp.int32),
            pltpu.VMEM((NCH, CHUNK, D), jnp.bfloat16),
            pltpu.VMEM((NCH, CHUNK, D), jnp.bfloat16),
            pltpu.SemaphoreType.DMA,
            pltpu.SemaphoreType.DMA,
        ],
        compiler_params=pltpu.CompilerParams(use_tc_tiling_on_sc=False),
    )
    def gather_sc16(sids, bids, bat, bowl, out_bat, out_bowl,
                    idx_a, idx_b, rows_a, rows_b, sem_a, sem_b):
        wid = lax.axis_index("s") * _NC + lax.axis_index("c")
        base = wid * B_PER_W
        for j in range(NCH):
            pltpu.sync_copy(sids.at[pl.ds(base + j * CHUNK, CHUNK)],
                            idx_a.at[j])
            pltpu.sync_copy(bids.at[pl.ds(base + j * CHUNK, CHUNK)],
                            idx_b.at[j])
        copies = []
        for j in range(NCH):
            copies.append(
                pltpu.async_copy(bat.at[idx_a.at[j]], rows_a.at[j], sem_a))
            copies.append(
                pltpu.async_copy(bowl.at[idx_b.at[j]], rows_b.at[j], sem_b))
        for c in copies:
            c.wait()
        for j in range(NCH):
            pltpu.sync_copy(rows_a.at[j],
                            out_bat.at[pl.ds(base + j * CHUNK, CHUNK)])
            pltpu.sync_copy(rows_b.at[j],
                            out_bowl.at[pl.ds(base + j * CHUNK, CHUNK)])

    return gather_sc16


BS = 2048


def _mlp_body(batg_ref, bowlg_ref, w1a_ref, w1b_ref, b1_ref, w2_ref, b2_ref,
              out_ref):
    a = jax.nn.sigmoid(batg_ref[...].astype(jnp.float32))
    b = jax.nn.sigmoid(bowlg_ref[...].astype(jnp.float32))
    h = jnp.dot(a, w1a_ref[...], preferred_element_type=jnp.float32)
    h = h + jnp.dot(b, w1b_ref[...], preferred_element_type=jnp.float32)
    h = jnp.maximum(h + b1_ref[...], 0.0)
    out_ref[...] = (
        jnp.dot(h, w2_ref[...], preferred_element_type=jnp.float32)
        + b2_ref[...])


def _mlp_tc(bat_g, bowl_g, w1a, w1b, b1r, w2t, b2r):
    return pl.pallas_call(
        _mlp_body,
        grid=(B // BS,),
        in_specs=[
            pl.BlockSpec((BS, D), lambda i: (i, 0)),
            pl.BlockSpec((BS, D), lambda i: (i, 0)),
            pl.BlockSpec((D, H), lambda i: (0, 0)),
            pl.BlockSpec((D, H), lambda i: (0, 0)),
            pl.BlockSpec((1, H), lambda i: (0, 0)),
            pl.BlockSpec((H, O), lambda i: (0, 0)),
            pl.BlockSpec((1, O), lambda i: (0, 0)),
        ],
        out_specs=pl.BlockSpec((BS, O), lambda i: (i, 0)),
        out_shape=jax.ShapeDtypeStruct((B, O), jnp.float32),
    )(bat_g, bowl_g, w1a, w1b, b1r, w2t, b2r)


def kernel(striker_ids, bowler_ids, bat_table, bowl_table, W1, b1, W2, b2):
    sids = striker_ids.astype(jnp.int32)
    bids = bowler_ids.astype(jnp.int32)
    bat_g, bowl_g = _build_gather_sc16()(
        sids, bids, bat_table.astype(jnp.bfloat16),
        bowl_table.astype(jnp.bfloat16))
    w1t = W1.T
    w1a = w1t[:D]
    w1b = w1t[D:]
    return _mlp_tc(bat_g, bowl_g, w1a, w1b,
                   b1.reshape(1, H), W2.T, b2.reshape(1, O))


# speedup vs baseline: 1.7183x; 1.7183x over previous
import functools

import jax
import jax.numpy as jnp
from jax import lax
from jax.experimental import pallas as pl
from jax.experimental.pallas import tpu as pltpu
from jax.experimental.pallas import tpu_sc as plsc

B = 16384
D = 16
H = 128
O = 32

_NC = 2
_NS = 16
_NW = _NC * _NS
B_PER_W = B // _NW
HALF = B_PER_W // 2
CHUNK = 128
L = 16
K = 64


@functools.cache
def _build_gather_sc():
    mesh = plsc.VectorSubcoreMesh(core_axis_name="c", subcore_axis_name="s")

    @functools.partial(
        pl.kernel,
        mesh=mesh,
        out_type=[
            jax.ShapeDtypeStruct((B, D), jnp.float32),
            jax.ShapeDtypeStruct((B, D), jnp.float32),
        ],
        scratch_types=[
            pltpu.VMEM((B_PER_W,), jnp.int32),
            pltpu.VMEM((B_PER_W,), jnp.int32),
            pltpu.VMEM((HALF, D), jnp.float32),
            pltpu.VMEM((HALF, D), jnp.float32),
            pltpu.SemaphoreType.DMA,
            pltpu.SemaphoreType.DMA,
        ],
        compiler_params=pltpu.CompilerParams(needs_layout_passes=False),
    )
    def gather_sc(sids, bids, bat, bowl, out_bat, out_bowl,
                  sid_v, bid_v, rows_a, rows_b, sem_a, sem_b):
        wid = lax.axis_index("s") * _NC + lax.axis_index("c")
        base = wid * B_PER_W
        pltpu.sync_copy(sids.at[pl.ds(base, B_PER_W)], sid_v)
        pltpu.sync_copy(bids.at[pl.ds(base, B_PER_W)], bid_v)

        for h in range(2):
            @pl.loop(0, HALF, step=K)
            def _(p0):
                copies = []
                for g in range(K // L):
                    qa = sid_v[pl.ds(h * HALF + p0 + g * L, L)]
                    qb = bid_v[pl.ds(h * HALF + p0 + g * L, L)]
                    for i in range(L):
                        p = p0 + g * L + i
                        copies.append(pltpu.async_copy(
                            bat.at[qa[i]], rows_a.at[p], sem_a))
                        copies.append(pltpu.async_copy(
                            bowl.at[qb[i]], rows_b.at[p], sem_b))
                for c in copies:
                    c.wait()

            for j in range(HALF // CHUNK):
                pltpu.sync_copy(
                    rows_a.at[pl.ds(j * CHUNK, CHUNK)],
                    out_bat.at[pl.ds(base + h * HALF + j * CHUNK, CHUNK)])
                pltpu.sync_copy(
                    rows_b.at[pl.ds(j * CHUNK, CHUNK)],
                    out_bowl.at[pl.ds(base + h * HALF + j * CHUNK, CHUNK)])

    return gather_sc


BS = 2048


def _mlp_body(batg_ref, bowlg_ref, w1a_ref, w1b_ref, b1_ref, w2_ref, b2_ref,
              out_ref):
    a = jax.nn.sigmoid(batg_ref[...])
    b = jax.nn.sigmoid(bowlg_ref[...])
    h = jnp.dot(a, w1a_ref[...], preferred_element_type=jnp.float32)
    h = h + jnp.dot(b, w1b_ref[...], preferred_element_type=jnp.float32)
    h = jnp.maximum(h + b1_ref[...], 0.0)
    out_ref[...] = (
        jnp.dot(h, w2_ref[...], preferred_element_type=jnp.float32)
        + b2_ref[...])


def _mlp_tc(bat_g, bowl_g, w1a, w1b, b1r, w2t, b2r):
    return pl.pallas_call(
        _mlp_body,
        grid=(B // BS,),
        in_specs=[
            pl.BlockSpec((BS, D), lambda i: (i, 0)),
            pl.BlockSpec((BS, D), lambda i: (i, 0)),
            pl.BlockSpec((D, H), lambda i: (0, 0)),
            pl.BlockSpec((D, H), lambda i: (0, 0)),
            pl.BlockSpec((1, H), lambda i: (0, 0)),
            pl.BlockSpec((H, O), lambda i: (0, 0)),
            pl.BlockSpec((1, O), lambda i: (0, 0)),
        ],
        out_specs=pl.BlockSpec((BS, O), lambda i: (i, 0)),
        out_shape=jax.ShapeDtypeStruct((B, O), jnp.float32),
    )(bat_g, bowl_g, w1a, w1b, b1r, w2t, b2r)


def kernel(striker_ids, bowler_ids, bat_table, bowl_table, W1, b1, W2, b2):
    sids = striker_ids.astype(jnp.int32)
    bids = bowler_ids.astype(jnp.int32)
    bat_g, bowl_g = _build_gather_sc()(sids, bids, bat_table, bowl_table)
    w1t = W1.T
    w1a = w1t[:D]
    w1b = w1t[D:]
    return _mlp_tc(bat_g, bowl_g, w1a, w1b,
                   b1.reshape(1, H), W2.T, b2.reshape(1, O))
